# initial kernel scaffold (unmeasured)
import jax
import jax.numpy as jnp
from jax import lax
from jax.experimental import pallas as pl
from jax.experimental.pallas import tpu as pltpu


def kernel(
    x,
):
    def body(*refs):
        pass

    out_shape = jax.ShapeDtypeStruct(..., jnp.float32)
    return pl.pallas_call(body, out_shape=out_shape)(...)


# baseline (device time: 374841 ns/iter reference)
import jax
import jax.numpy as jnp
from jax import lax
from jax.experimental import pallas as pl
from jax.experimental.pallas import tpu as pltpu

N_DEV = 8


def _mod(v):
    return lax.rem(v + 2 * N_DEV, N_DEV)


def kernel(x):
    m, n = x.shape
    ch = m // N_DEV
    xb = x.astype(jnp.bfloat16)

    def body(x_ref, out_ref, comm_ref, rs_send_sems, rs_recv_sems,
             ag_send_sems, ag_recv_sems):
        me = lax.axis_index("i")
        right = _mod(me + 1)
        left = _mod(me - 1)

        barrier_sem = pltpu.get_barrier_semaphore()
        for nbr in (left, right):
            pl.semaphore_signal(
                barrier_sem, inc=1,
                device_id=(nbr,), device_id_type=pl.DeviceIdType.MESH,
            )
        pl.semaphore_wait(barrier_sem, 2)

        out_ref[...] = x_ref[...]

        for s in range(N_DEV - 1):
            c_send = _mod(me - s)
            c_recv = _mod(me - s - 1)
            rdma = pltpu.make_async_remote_copy(
                src_ref=out_ref.at[pl.ds(c_send * ch, ch), :],
                dst_ref=comm_ref.at[s],
                send_sem=rs_send_sems.at[s],
                recv_sem=rs_recv_sems.at[s],
                device_id=(right,),
                device_id_type=pl.DeviceIdType.MESH,
            )
            rdma.start()
            rdma.wait()
            out_ref[pl.ds(c_recv * ch, ch), :] = (
                out_ref[pl.ds(c_recv * ch, ch), :] + comm_ref[s]
            )

        for t in range(N_DEV - 1):
            c_send = _mod(me + 1 - t)
            rdma = pltpu.make_async_remote_copy(
                src_ref=out_ref.at[pl.ds(c_send * ch, ch), :],
                dst_ref=out_ref.at[pl.ds(c_send * ch, ch), :],
                send_sem=ag_send_sems.at[t],
                recv_sem=ag_recv_sems.at[t],
                device_id=(right,),
                device_id_type=pl.DeviceIdType.MESH,
            )
            rdma.start()
            rdma.wait()

    return pl.pallas_call(
        body,
        out_shape=jax.ShapeDtypeStruct((m, n), jnp.bfloat16),
        in_specs=[pl.BlockSpec(memory_space=pltpu.VMEM)],
        out_specs=pl.BlockSpec(memory_space=pltpu.VMEM),
        scratch_shapes=[
            pltpu.VMEM((N_DEV - 1, ch, n), jnp.bfloat16),
            pltpu.SemaphoreType.DMA((N_DEV - 1,)),
            pltpu.SemaphoreType.DMA((N_DEV - 1,)),
            pltpu.SemaphoreType.DMA((N_DEV - 1,)),
            pltpu.SemaphoreType.DMA((N_DEV - 1,)),
        ],
        compiler_params=pltpu.CompilerParams(collective_id=0),
    )(xb)


# device time: 223510 ns/iter; 1.6771x vs baseline; 1.6771x over previous
import jax
import jax.numpy as jnp
from jax import lax
from jax.experimental import pallas as pl
from jax.experimental.pallas import tpu as pltpu

N_DEV = 8


def _mod(v):
    return lax.rem(v + 2 * N_DEV, N_DEV)


def kernel(x):
    m, n = x.shape
    ch = m // N_DEV
    hn = n // 2
    xb = x.astype(jnp.bfloat16)

    def body(x_ref, out_ref, comm_r, comm_l,
             rs_send_r, rs_recv_r, rs_send_l, rs_recv_l,
             ag_send_r, ag_recv_r, ag_send_l, ag_recv_l):
        me = lax.axis_index("i")
        right = _mod(me + 1)
        left = _mod(me - 1)

        barrier_sem = pltpu.get_barrier_semaphore()
        for nbr in (left, right):
            pl.semaphore_signal(
                barrier_sem, inc=1,
                device_id=(nbr,), device_id_type=pl.DeviceIdType.MESH,
            )
        pl.semaphore_wait(barrier_sem, 2)

        out_ref[...] = x_ref[...]

        for s in range(N_DEV - 1):
            cs_r = _mod(me - s)
            cr_r = _mod(me - s - 1)
            cs_l = _mod(me + s)
            cr_l = _mod(me + s + 1)
            rdma_r = pltpu.make_async_remote_copy(
                src_ref=out_ref.at[pl.ds(cs_r * ch, ch), 0:hn],
                dst_ref=comm_r.at[s],
                send_sem=rs_send_r.at[s],
                recv_sem=rs_recv_r.at[s],
                device_id=(right,),
                device_id_type=pl.DeviceIdType.MESH,
            )
            rdma_l = pltpu.make_async_remote_copy(
                src_ref=out_ref.at[pl.ds(cs_l * ch, ch), hn:n],
                dst_ref=comm_l.at[s],
                send_sem=rs_send_l.at[s],
                recv_sem=rs_recv_l.at[s],
                device_id=(left,),
                device_id_type=pl.DeviceIdType.MESH,
            )
            rdma_r.start()
            rdma_l.start()
            rdma_r.wait()
            rdma_l.wait()
            out_ref[pl.ds(cr_r * ch, ch), 0:hn] = (
                out_ref[pl.ds(cr_r * ch, ch), 0:hn] + comm_r[s]
            )
            out_ref[pl.ds(cr_l * ch, ch), hn:n] = (
                out_ref[pl.ds(cr_l * ch, ch), hn:n] + comm_l[s]
            )

        for t in range(N_DEV - 1):
            cs_r = _mod(me + 1 - t)
            cs_l = _mod(me - 1 + t)
            rdma_r = pltpu.make_async_remote_copy(
                src_ref=out_ref.at[pl.ds(cs_r * ch, ch), 0:hn],
                dst_ref=out_ref.at[pl.ds(cs_r * ch, ch), 0:hn],
                send_sem=ag_send_r.at[t],
                recv_sem=ag_recv_r.at[t],
                device_id=(right,),
                device_id_type=pl.DeviceIdType.MESH,
            )
            rdma_l = pltpu.make_async_remote_copy(
                src_ref=out_ref.at[pl.ds(cs_l * ch, ch), hn:n],
                dst_ref=out_ref.at[pl.ds(cs_l * ch, ch), hn:n],
                send_sem=ag_send_l.at[t],
                recv_sem=ag_recv_l.at[t],
                device_id=(left,),
                device_id_type=pl.DeviceIdType.MESH,
            )
            rdma_r.start()
            rdma_l.start()
            rdma_r.wait()
            rdma_l.wait()

    return pl.pallas_call(
        body,
        out_shape=jax.ShapeDtypeStruct((m, n), jnp.bfloat16),
        in_specs=[pl.BlockSpec(memory_space=pltpu.VMEM)],
        out_specs=pl.BlockSpec(memory_space=pltpu.VMEM),
        scratch_shapes=[
            pltpu.VMEM((N_DEV - 1, ch, hn), jnp.bfloat16),
            pltpu.VMEM((N_DEV - 1, ch, hn), jnp.bfloat16),
            pltpu.SemaphoreType.DMA((N_DEV - 1,)),
            pltpu.SemaphoreType.DMA((N_DEV - 1,)),
            pltpu.SemaphoreType.DMA((N_DEV - 1,)),
            pltpu.SemaphoreType.DMA((N_DEV - 1,)),
            pltpu.SemaphoreType.DMA((N_DEV - 1,)),
            pltpu.SemaphoreType.DMA((N_DEV - 1,)),
            pltpu.SemaphoreType.DMA((N_DEV - 1,)),
            pltpu.SemaphoreType.DMA((N_DEV - 1,)),
        ],
        compiler_params=pltpu.CompilerParams(collective_id=0),
    )(xb)


# device time: 217659 ns/iter; 1.7221x vs baseline; 1.0269x over previous
import jax
import jax.numpy as jnp
from jax import lax
from jax.experimental import pallas as pl
from jax.experimental.pallas import tpu as pltpu

N_DEV = 8


def _mod(v):
    return lax.rem(v + 2 * N_DEV, N_DEV)


def kernel(x):
    m, n = x.shape
    ch = m // N_DEV
    hn = n // 2
    xb = x.astype(jnp.bfloat16)

    def body(x_ref, out_ref, comm_r, comm_l,
             rs_send_r, rs_recv_r, rs_send_l, rs_recv_l,
             ag_send_r, ag_recv_r, ag_send_l, ag_recv_l):
        me = lax.axis_index("i")
        right = _mod(me + 1)
        left = _mod(me - 1)

        def rs_r(s):
            cs = _mod(me - s)
            src = x_ref if s == 0 else out_ref
            return pltpu.make_async_remote_copy(
                src_ref=src.at[pl.ds(cs * ch, ch), 0:hn],
                dst_ref=comm_r.at[s],
                send_sem=rs_send_r.at[s],
                recv_sem=rs_recv_r.at[s],
                device_id=(right,),
                device_id_type=pl.DeviceIdType.MESH,
            )

        def rs_l(s):
            cs = _mod(me + s)
            src = x_ref if s == 0 else out_ref
            return pltpu.make_async_remote_copy(
                src_ref=src.at[pl.ds(cs * ch, ch), hn:n],
                dst_ref=comm_l.at[s],
                send_sem=rs_send_l.at[s],
                recv_sem=rs_recv_l.at[s],
                device_id=(left,),
                device_id_type=pl.DeviceIdType.MESH,
            )

        def ag_r(t):
            cs = _mod(me + 1 - t)
            return pltpu.make_async_remote_copy(
                src_ref=out_ref.at[pl.ds(cs * ch, ch), 0:hn],
                dst_ref=out_ref.at[pl.ds(cs * ch, ch), 0:hn],
                send_sem=ag_send_r.at[t],
                recv_sem=ag_recv_r.at[t],
                device_id=(right,),
                device_id_type=pl.DeviceIdType.MESH,
            )

        def ag_l(t):
            cs = _mod(me - 1 + t)
            return pltpu.make_async_remote_copy(
                src_ref=out_ref.at[pl.ds(cs * ch, ch), hn:n],
                dst_ref=out_ref.at[pl.ds(cs * ch, ch), hn:n],
                send_sem=ag_send_l.at[t],
                recv_sem=ag_recv_l.at[t],
                device_id=(left,),
                device_id_type=pl.DeviceIdType.MESH,
            )

        barrier_sem = pltpu.get_barrier_semaphore()
        for nbr in (left, right):
            pl.semaphore_signal(
                barrier_sem, inc=1,
                device_id=(nbr,), device_id_type=pl.DeviceIdType.MESH,
            )
        pl.semaphore_wait(barrier_sem, 2)

        rs_r(0).start()
        rs_l(0).start()
        for s in range(N_DEV - 1):
            cr_r = _mod(me - s - 1)
            cr_l = _mod(me + s + 1)
            rs_r(s).wait_recv()
            out_ref[pl.ds(cr_r * ch, ch), 0:hn] = (
                x_ref[pl.ds(cr_r * ch, ch), 0:hn] + comm_r[s]
            )
            if s < N_DEV - 2:
                rs_r(s + 1).start()
            rs_l(s).wait_recv()
            out_ref[pl.ds(cr_l * ch, ch), hn:n] = (
                x_ref[pl.ds(cr_l * ch, ch), hn:n] + comm_l[s]
            )
            if s < N_DEV - 2:
                rs_l(s + 1).start()

        ag_r(0).start()
        ag_l(0).start()
        for t in range(N_DEV - 1):
            ag_r(t).wait_recv()
            if t < N_DEV - 2:
                ag_r(t + 1).start()
            ag_l(t).wait_recv()
            if t < N_DEV - 2:
                ag_l(t + 1).start()

        for s in range(N_DEV - 1):
            rs_r(s).wait_send()
            rs_l(s).wait_send()
            ag_r(s).wait_send()
            ag_l(s).wait_send()

    return pl.pallas_call(
        body,
        out_shape=jax.ShapeDtypeStruct((m, n), jnp.bfloat16),
        in_specs=[pl.BlockSpec(memory_space=pltpu.VMEM)],
        out_specs=pl.BlockSpec(memory_space=pltpu.VMEM),
        scratch_shapes=[
            pltpu.VMEM((N_DEV - 1, ch, hn), jnp.bfloat16),
            pltpu.VMEM((N_DEV - 1, ch, hn), jnp.bfloat16),
            pltpu.SemaphoreType.DMA((N_DEV - 1,)),
            pltpu.SemaphoreType.DMA((N_DEV - 1,)),
            pltpu.SemaphoreType.DMA((N_DEV - 1,)),
            pltpu.SemaphoreType.DMA((N_DEV - 1,)),
            pltpu.SemaphoreType.DMA((N_DEV - 1,)),
            pltpu.SemaphoreType.DMA((N_DEV - 1,)),
            pltpu.SemaphoreType.DMA((N_DEV - 1,)),
            pltpu.SemaphoreType.DMA((N_DEV - 1,)),
        ],
        compiler_params=pltpu.CompilerParams(collective_id=0),
    )(xb)


# device time: 195807 ns/iter; 1.9143x vs baseline; 1.1116x over previous
import jax
import jax.numpy as jnp
from jax import lax
from jax.experimental import pallas as pl
from jax.experimental.pallas import tpu as pltpu

N_DEV = 8
N_SUB = 2


def _mod(v):
    return lax.rem(v + 2 * N_DEV, N_DEV)


def kernel(x):
    m, n = x.shape
    ch = m // N_DEV
    hn = n // 2
    hr = ch // N_SUB
    xb = x.astype(jnp.bfloat16)

    def body(x_ref, out_ref, comm_r, comm_l,
             rs_send_r, rs_recv_r, rs_send_l, rs_recv_l,
             ag_send_r, ag_recv_r, ag_send_l, ag_recv_l):
        me = lax.axis_index("i")
        right = _mod(me + 1)
        left = _mod(me - 1)

        def rs_r(s, j):
            cs = _mod(me - s)
            src = x_ref if s == 0 else out_ref
            return pltpu.make_async_remote_copy(
                src_ref=src.at[pl.ds(cs * ch + j * hr, hr), 0:hn],
                dst_ref=comm_r.at[s, pl.ds(j * hr, hr), :],
                send_sem=rs_send_r.at[s, j],
                recv_sem=rs_recv_r.at[s, j],
                device_id=(right,),
                device_id_type=pl.DeviceIdType.MESH,
            )

        def rs_l(s, j):
            cs = _mod(me + s)
            src = x_ref if s == 0 else out_ref
            return pltpu.make_async_remote_copy(
                src_ref=src.at[pl.ds(cs * ch + j * hr, hr), hn:n],
                dst_ref=comm_l.at[s, pl.ds(j * hr, hr), :],
                send_sem=rs_send_l.at[s, j],
                recv_sem=rs_recv_l.at[s, j],
                device_id=(left,),
                device_id_type=pl.DeviceIdType.MESH,
            )

        def ag_r(t, j):
            cs = _mod(me + 1 - t)
            sl = (pl.ds(cs * ch + j * hr, hr), slice(0, hn))
            return pltpu.make_async_remote_copy(
                src_ref=out_ref.at[sl],
                dst_ref=out_ref.at[sl],
                send_sem=ag_send_r.at[t, j],
                recv_sem=ag_recv_r.at[t, j],
                device_id=(right,),
                device_id_type=pl.DeviceIdType.MESH,
            )

        def ag_l(t, j):
            cs = _mod(me - 1 + t)
            sl = (pl.ds(cs * ch + j * hr, hr), slice(hn, n))
            return pltpu.make_async_remote_copy(
                src_ref=out_ref.at[sl],
                dst_ref=out_ref.at[sl],
                send_sem=ag_send_l.at[t, j],
                recv_sem=ag_recv_l.at[t, j],
                device_id=(left,),
                device_id_type=pl.DeviceIdType.MESH,
            )

        barrier_sem = pltpu.get_barrier_semaphore()
        for nbr in (left, right):
            pl.semaphore_signal(
                barrier_sem, inc=1,
                device_id=(nbr,), device_id_type=pl.DeviceIdType.MESH,
            )
        pl.semaphore_wait(barrier_sem, 2)

        for j in range(N_SUB):
            rs_r(0, j).start()
            rs_l(0, j).start()
        for s in range(N_DEV - 1):
            cr_r = _mod(me - s - 1)
            cr_l = _mod(me + s + 1)
            for j in range(N_SUB):
                row_r = pl.ds(cr_r * ch + j * hr, hr)
                rs_r(s, j).wait_recv()
                out_ref[row_r, 0:hn] = (
                    x_ref[row_r, 0:hn]
                    + comm_r[s, pl.ds(j * hr, hr), :]
                )
                if s < N_DEV - 2:
                    rs_r(s + 1, j).start()
                row_l = pl.ds(cr_l * ch + j * hr, hr)
                rs_l(s, j).wait_recv()
                out_ref[row_l, hn:n] = (
                    x_ref[row_l, hn:n]
                    + comm_l[s, pl.ds(j * hr, hr), :]
                )
                if s < N_DEV - 2:
                    rs_l(s + 1, j).start()

        for j in range(N_SUB):
            ag_r(0, j).start()
            ag_l(0, j).start()
        for t in range(N_DEV - 1):
            for j in range(N_SUB):
                ag_r(t, j).wait_recv()
                if t < N_DEV - 2:
                    ag_r(t + 1, j).start()
                ag_l(t, j).wait_recv()
                if t < N_DEV - 2:
                    ag_l(t + 1, j).start()

        for s in range(N_DEV - 1):
            for j in range(N_SUB):
                rs_r(s, j).wait_send()
                rs_l(s, j).wait_send()
                ag_r(s, j).wait_send()
                ag_l(s, j).wait_send()

    sems = pltpu.SemaphoreType.DMA((N_DEV - 1, N_SUB))
    return pl.pallas_call(
        body,
        out_shape=jax.ShapeDtypeStruct((m, n), jnp.bfloat16),
        in_specs=[pl.BlockSpec(memory_space=pltpu.VMEM)],
        out_specs=pl.BlockSpec(memory_space=pltpu.VMEM),
        scratch_shapes=[
            pltpu.VMEM((N_DEV - 1, ch, hn), jnp.bfloat16),
            pltpu.VMEM((N_DEV - 1, ch, hn), jnp.bfloat16),
            sems, sems, sems, sems,
            sems, sems, sems, sems,
        ],
        compiler_params=pltpu.CompilerParams(collective_id=0),
    )(xb)
